# baseline (speedup 1.0000x reference)
import functools

import jax
import jax.numpy as jnp
from jax import lax
from jax.experimental import pallas as pl
from jax.experimental.pallas import tpu as pltpu
from jax.experimental.pallas import tpu_sc as plsc

_H = 52
_SUB = 4


def kernel(features, is_start):
    del is_start
    xt = jnp.transpose(features, (2, 3, 0, 1))
    h, w, b, c = xt.shape
    info = plsc.get_sparse_core_info()
    nw = info.num_cores * info.num_subcores
    step = w // _SUB

    @functools.partial(
        pl.kernel,
        mesh=plsc.VectorSubcoreMesh(core_axis_name="c", subcore_axis_name="s"),
        out_type=jax.ShapeDtypeStruct(xt.shape, xt.dtype),
        scratch_types=[pltpu.VMEM((step, b, c), xt.dtype)],
    )
    def _sc_copy(x_hbm, o_hbm, buf):
        wid = lax.axis_index("s") * info.num_cores + lax.axis_index("c")
        for i in range((h + nw - 1) // nw):
            r = wid + i * nw
            @pl.when(r < h)
            def _():
                for ci in range(_SUB):
                    pltpu.sync_copy(x_hbm.at[r, pl.ds(ci * step, step)], buf)
                    pltpu.sync_copy(buf, o_hbm.at[r, pl.ds(ci * step, step)])

    out = _sc_copy(xt)
    return jnp.transpose(out, (2, 3, 0, 1))


# speedup vs baseline: 3.0182x; 3.0182x over previous
import jax
import jax.numpy as jnp
from jax.experimental import pallas as pl
from jax.experimental.pallas import tpu as pltpu

_GRID = 2


def _copy_body(x_ref, o_ref):
    o_ref[...] = x_ref[...]


def kernel(features, is_start):
    del is_start
    xt = jnp.transpose(features, (2, 3, 0, 1))
    h, w, b, c = xt.shape
    step = h // _GRID
    out = pl.pallas_call(
        _copy_body,
        grid=(_GRID,),
        in_specs=[pl.BlockSpec((step, w, b, c), lambda i: (i, 0, 0, 0))],
        out_specs=pl.BlockSpec((step, w, b, c), lambda i: (i, 0, 0, 0)),
        out_shape=jax.ShapeDtypeStruct(xt.shape, xt.dtype),
        compiler_params=pltpu.CompilerParams(
            dimension_semantics=("arbitrary",),
        ),
    )(xt)
    return jnp.transpose(out, (2, 3, 0, 1))
